# baseline (speedup 1.0000x reference)
import functools

import jax
import jax.numpy as jnp
from jax import lax
from jax.experimental import pallas as pl
from jax.experimental.pallas import tpu as pltpu
from jax.experimental.pallas import tpu_sc as plsc

HIDDEN = 64
SEQ = 200
CHUNK = 400
SUB = 50
NSUB = CHUNK // SUB
SUB2 = 80
NSUB2 = CHUNK // SUB2
NSEQ = CHUNK // SEQ
TPAD = 208
NWORKERS = 32
EPS = 1e-5


def _rsqrt16(x):
    i = lax.bitcast_convert_type(x, jnp.int32)
    y = lax.bitcast_convert_type(jnp.int32(0x5F3759DF) - (i >> 1), jnp.float32)
    hx = x * (-0.5)
    for _ in range(3):
        y = y * (1.5 + hx * y * y)
    return y


@functools.partial(jax.jit, static_argnames=("batch", "seq"))
def _run(token_ids, type_ids, token_table, type_table, pos_table, ln_gamma,
         ln_beta, *, batch, seq):
    flat = batch * seq
    tpw = flat // NWORKERS
    nchunk = tpw // CHUNK

    tok2 = token_ids.reshape(flat // SUB, SUB).astype(jnp.int32)

    mesh = plsc.VectorSubcoreMesh(core_axis_name="c", subcore_axis_name="s")

    @functools.partial(
        pl.kernel,
        mesh=mesh,
        compiler_params=pltpu.CompilerParams(use_tc_tiling_on_sc=False,
                                             needs_layout_passes=False),
        out_type=(jax.ShapeDtypeStruct((batch, seq, HIDDEN), jnp.float32),
                  jax.ShapeDtypeStruct((2 * 4 * TPAD, HIDDEN), jnp.float32)),
        scratch_types=[
            pltpu.VMEM((NSUB, SUB), jnp.int32),
            pltpu.VMEM((NSUB, SUB), jnp.int32),
            pltpu.VMEM((CHUNK,), jnp.int32),
            pltpu.VMEM((CHUNK,), jnp.int32),
            pltpu.VMEM((CHUNK,), jnp.int32),
            pltpu.VMEM((CHUNK,), jnp.int32),
            pltpu.VMEM((CHUNK, HIDDEN), jnp.float32),
            pltpu.VMEM((CHUNK, HIDDEN), jnp.float32),
            pltpu.VMEM((CHUNK, HIDDEN), jnp.float32),
            pltpu.VMEM((CHUNK, HIDDEN), jnp.float32),
            pltpu.VMEM((4, HIDDEN), jnp.float32),
            pltpu.VMEM((HIDDEN,), jnp.float32),
            pltpu.VMEM((HIDDEN,), jnp.float32),
            pltpu.SemaphoreType.DMA,
            pltpu.SemaphoreType.DMA,
            pltpu.SemaphoreType.DMA,
            pltpu.SemaphoreType.DMA,
        ],
    )
    def sc_kernel(tok_hbm, typ_hbm, table_hbm, type_t_hbm, pos_hbm, g_hbm,
                  b_hbm, out_hbm, ct_hbm, idx0, idx1, tixv0, tixv1, cidx0,
                  cidx1, rows0, rows1, trows0, trows1, typb,
                  gv, bv, gsem0, gsem1, ssem0, ssem1):
        wid = lax.axis_index("s") * 2 + lax.axis_index("c")
        base = wid * tpw
        srow = base // SEQ

        pltpu.sync_copy(type_t_hbm, typb)
        pltpu.sync_copy(g_hbm, gv)
        pltpu.sync_copy(b_hbm, bv)

        g = [gv[pl.ds(k * 16, 16)] for k in range(4)]
        b = [bv[pl.ds(k * 16, 16)] for k in range(4)]
        lanes = lax.iota(jnp.int32, 16)
        bfly = [lanes ^ sh for sh in (1, 2, 4, 8)]

        coff = lax.axis_index("c") * (4 * TPAD)

        @pl.when(lax.axis_index("s") == 0)
        def _():
            pltpu.sync_copy(pos_hbm.at[pl.ds(0, SEQ)],
                            trows0.at[pl.ds(0, SEQ)])
            for tid in range(4):
                t = [typb[tid, pl.ds(k * 16, 16)] for k in range(4)]

                def bld(p, _):
                    for k in range(4):
                        trows0[SEQ + p, pl.ds(k * 16, 16)] = (
                            trows0[p, pl.ds(k * 16, 16)] + t[k])
                    return 0

                lax.fori_loop(0, SEQ, bld, 0, unroll=4)
                pltpu.sync_copy(
                    trows0.at[pl.ds(SEQ, SEQ)],
                    ct_hbm.at[pl.ds(pl.multiple_of(coff + tid * TPAD, 8),
                                    SEQ)])

        plsc.subcore_barrier()

        idx = (idx0, idx1)
        tixv = (tixv0, tixv1)
        cidx = (cidx0, cidx1)
        rows = (rows0, rows1)
        trows = (trows0, trows1)
        gsem = (gsem0, gsem1)
        ssem = (ssem0, ssem1)

        def start_gather(c, bi):
            cbase = pl.multiple_of(base + c * CHUNK, CHUNK)
            pltpu.sync_copy(
                tok_hbm.at[pl.ds(pl.multiple_of(cbase // SUB, NSUB), NSUB)],
                idx[bi])
            pltpu.sync_copy(typ_hbm.at[pl.ds(cbase, CHUNK)], tixv[bi])
            for j in range(NSUB):
                pltpu.async_copy(
                    table_hbm.at[idx[bi].at[j]],
                    rows[bi].at[pl.ds(j * SUB, SUB)],
                    gsem[bi],
                )
            tv = tixv[bi]
            cv = cidx[bi]

            def cix(gi, pv):
                cv[pl.ds(gi * 16, 16)] = (tv[pl.ds(gi * 16, 16)] * TPAD
                                          + (pv + coff))
                pn = pv + 16
                return jnp.where(pn >= SEQ, pn - SEQ, pn)

            lax.fori_loop(0, CHUNK // 16, cix, lanes, unroll=4)
            for j in range(NSUB2):
                pltpu.async_copy(
                    ct_hbm.at[cv.at[pl.ds(j * SUB2, SUB2)]],
                    trows[bi].at[pl.ds(j * SUB2, SUB2)],
                    gsem[bi],
                )

        def wait_gather(bi):
            pltpu.make_async_copy(
                table_hbm.at[pl.ds(0, CHUNK)], rows[bi], gsem[bi]
            ).wait()
            pltpu.make_async_copy(
                table_hbm.at[pl.ds(0, CHUNK)], trows[bi], gsem[bi]
            ).wait()

        def start_scatter(c, bi):
            for s in range(NSEQ):
                pltpu.async_copy(
                    trows[bi].at[pl.ds(s * SEQ, SEQ)],
                    out_hbm.at[srow + c * NSEQ + s],
                    ssem[bi],
                )

        def wait_scatter(bi):
            pltpu.make_async_copy(
                trows[bi], table_hbm.at[pl.ds(0, CHUNK)], ssem[bi]
            ).wait()

        def compute_chunk(bi):
            rbuf = rows[bi]
            tbuf = trows[bi]
            ob = trows[bi]

            def tok_body(i, _):
                y = [
                    rbuf[i, pl.ds(k * 16, 16)] + tbuf[i, pl.ds(k * 16, 16)]
                    for k in range(4)
                ]
                s = (y[0] + y[1]) + (y[2] + y[3])
                q = (y[0] * y[0] + y[1] * y[1]) + (y[2] * y[2] + y[3] * y[3])
                for perm in bfly:
                    s = s + s.at[perm].get(mode="promise_in_bounds")
                    q = q + q.at[perm].get(mode="promise_in_bounds")
                mv = s * (1.0 / HIDDEN)
                var = q * (1.0 / HIDDEN) - mv * mv
                inv = _rsqrt16(var + EPS)
                for k in range(4):
                    ob[i, pl.ds(k * 16, 16)] = (y[k] - mv) * inv * g[k] + b[k]
                return 0

            lax.fori_loop(0, CHUNK, tok_body, 0, unroll=16)

        start_gather(0, 0)

        def outer(co, _):
            for bstat in range(2):
                c = co * 2 + bstat
                nb = 1 - bstat

                @pl.when(c >= 1)
                def _():
                    wait_scatter(nb)

                @pl.when(c + 1 < nchunk)
                def _():
                    start_gather(c + 1, nb)

                wait_gather(bstat)
                compute_chunk(bstat)
                start_scatter(c, bstat)
            return 0

        lax.fori_loop(0, nchunk // 2, outer, 0)
        wait_scatter(1)

    out, _ = sc_kernel(tok2, type_ids.reshape(flat).astype(jnp.int32),
                       token_table, type_table, pos_table, ln_gamma, ln_beta)
    return out


def kernel(token_ids, type_ids, token_table, type_table, pos_table, ln_gamma,
           ln_beta):
    batch, seq = token_ids.shape
    return _run(token_ids, type_ids, token_table, type_table, pos_table,
                ln_gamma, ln_beta, batch=batch, seq=seq)


# speedup vs baseline: 1.4456x; 1.4456x over previous
import functools

import jax
import jax.numpy as jnp
from jax import lax
from jax.experimental import pallas as pl
from jax.experimental.pallas import tpu as pltpu
from jax.experimental.pallas import tpu_sc as plsc

HIDDEN = 64
SEQ = 200
CHUNK = 400
SUB = 50
NSUB = CHUNK // SUB
SUB2 = 80
NSUB2 = CHUNK // SUB2
NSEQ = CHUNK // SEQ
TPAD = 208
NWORKERS = 32
EPS = 1e-5


def _rsqrt16(x):
    i = lax.bitcast_convert_type(x, jnp.int32)
    y = lax.bitcast_convert_type(jnp.int32(0x5F3759DF) - (i >> 1), jnp.float32)
    hx = x * (-0.5)
    for _ in range(3):
        y = y * (1.5 + hx * y * y)
    return y


@functools.partial(jax.jit, static_argnames=("batch", "seq"))
def _run(token_ids, type_ids, token_table, type_table, pos_table, ln_gamma,
         ln_beta, *, batch, seq):
    flat = batch * seq
    tpw = flat // NWORKERS
    nchunk = tpw // CHUNK

    tok2 = token_ids.reshape(flat // SUB, SUB).astype(jnp.int32)

    mesh = plsc.VectorSubcoreMesh(core_axis_name="c", subcore_axis_name="s")

    @functools.partial(
        pl.kernel,
        mesh=mesh,
        compiler_params=pltpu.CompilerParams(use_tc_tiling_on_sc=False,
                                             needs_layout_passes=False),
        out_type=(jax.ShapeDtypeStruct((batch, seq, HIDDEN), jnp.float32),
                  jax.ShapeDtypeStruct((2 * 4 * TPAD, HIDDEN), jnp.float32)),
        scratch_types=[
            pltpu.VMEM((NSUB, SUB), jnp.int32),
            pltpu.VMEM((NSUB, SUB), jnp.int32),
            pltpu.VMEM((CHUNK,), jnp.int32),
            pltpu.VMEM((CHUNK,), jnp.int32),
            pltpu.VMEM((CHUNK,), jnp.int32),
            pltpu.VMEM((CHUNK,), jnp.int32),
            pltpu.VMEM((CHUNK, HIDDEN), jnp.float32),
            pltpu.VMEM((CHUNK, HIDDEN), jnp.float32),
            pltpu.VMEM((CHUNK, HIDDEN), jnp.float32),
            pltpu.VMEM((CHUNK, HIDDEN), jnp.float32),
            pltpu.VMEM((4, HIDDEN), jnp.float32),
            pltpu.VMEM((HIDDEN,), jnp.float32),
            pltpu.VMEM((HIDDEN,), jnp.float32),
            pltpu.SemaphoreType.DMA,
            pltpu.SemaphoreType.DMA,
            pltpu.SemaphoreType.DMA,
            pltpu.SemaphoreType.DMA,
        ],
    )
    def sc_kernel(tok_hbm, typ_hbm, table_hbm, type_t_hbm, pos_hbm, g_hbm,
                  b_hbm, out_hbm, ct_hbm, idx0, idx1, tixv0, tixv1, cidx0,
                  cidx1, rows0, rows1, trows0, trows1, typb,
                  gv, bv, gsem0, gsem1, ssem0, ssem1):
        wid = lax.axis_index("s") * 2 + lax.axis_index("c")
        base = wid * tpw
        srow = base // SEQ

        pltpu.sync_copy(type_t_hbm, typb)
        pltpu.sync_copy(g_hbm, gv)
        pltpu.sync_copy(b_hbm, bv)

        g = [gv[pl.ds(k * 16, 16)] for k in range(4)]
        b = [bv[pl.ds(k * 16, 16)] for k in range(4)]
        lanes = lax.iota(jnp.int32, 16)
        bfly = [lanes ^ sh for sh in (1, 2, 4, 8)]

        coff = lax.axis_index("c") * (4 * TPAD)

        @pl.when(lax.axis_index("s") == 0)
        def _():
            pltpu.sync_copy(pos_hbm.at[pl.ds(0, SEQ)],
                            trows0.at[pl.ds(0, SEQ)])
            for tid in range(4):
                t = [typb[tid, pl.ds(k * 16, 16)] for k in range(4)]

                def bld(p, _):
                    for k in range(4):
                        trows0[SEQ + p, pl.ds(k * 16, 16)] = (
                            trows0[p, pl.ds(k * 16, 16)] + t[k])
                    return 0

                lax.fori_loop(0, SEQ, bld, 0, unroll=4)
                pltpu.sync_copy(
                    trows0.at[pl.ds(SEQ, SEQ)],
                    ct_hbm.at[pl.ds(pl.multiple_of(coff + tid * TPAD, 8),
                                    SEQ)])

        plsc.subcore_barrier()

        idx = (idx0, idx1)
        tixv = (tixv0, tixv1)
        cidx = (cidx0, cidx1)
        rows = (rows0, rows1)
        trows = (trows0, trows1)
        gsem = (gsem0, gsem1)
        ssem = (ssem0, ssem1)

        def start_gather(c, bi):
            cbase = pl.multiple_of(base + c * CHUNK, CHUNK)
            pltpu.sync_copy(
                tok_hbm.at[pl.ds(pl.multiple_of(cbase // SUB, NSUB), NSUB)],
                idx[bi])
            pltpu.sync_copy(typ_hbm.at[pl.ds(cbase, CHUNK)], tixv[bi])
            for j in range(NSUB):
                pltpu.async_copy(
                    table_hbm.at[idx[bi].at[j]],
                    rows[bi].at[pl.ds(j * SUB, SUB)],
                    gsem[bi],
                )
            tv = tixv[bi]
            cv = cidx[bi]

            def cix(gi, pv):
                cv[pl.ds(gi * 16, 16)] = (tv[pl.ds(gi * 16, 16)] * TPAD
                                          + (pv + coff))
                pn = pv + 16
                return jnp.where(pn >= SEQ, pn - SEQ, pn)

            lax.fori_loop(0, CHUNK // 16, cix, lanes, unroll=4)
            for j in range(NSUB2):
                pltpu.async_copy(
                    ct_hbm.at[cv.at[pl.ds(j * SUB2, SUB2)]],
                    trows[bi].at[pl.ds(j * SUB2, SUB2)],
                    gsem[bi],
                )

        def wait_gather(bi):
            pltpu.make_async_copy(
                table_hbm.at[pl.ds(0, CHUNK)], rows[bi], gsem[bi]
            ).wait()
            pltpu.make_async_copy(
                table_hbm.at[pl.ds(0, CHUNK)], trows[bi], gsem[bi]
            ).wait()

        def start_scatter(c, bi):
            for s in range(NSEQ):
                pltpu.async_copy(
                    trows[bi].at[pl.ds(s * SEQ, SEQ)],
                    out_hbm.at[srow + c * NSEQ + s],
                    ssem[bi],
                )

        def wait_scatter(bi):
            pltpu.make_async_copy(
                trows[bi], table_hbm.at[pl.ds(0, CHUNK)], ssem[bi]
            ).wait()

        def compute_chunk(bi):
            rbuf = rows[bi]
            tbuf = trows[bi]
            ob = trows[bi]

            def grp_body(gi, _):
                i0 = gi * 16
                pmv = jnp.zeros((16,), jnp.float32)
                pvr = jnp.zeros((16,), jnp.float32)
                for j in range(16):
                    i = i0 + j
                    y = [
                        rbuf[i, pl.ds(k * 16, 16)]
                        + tbuf[i, pl.ds(k * 16, 16)]
                        for k in range(4)
                    ]
                    s = (y[0] + y[1]) + (y[2] + y[3])
                    q = (y[0] * y[0] + y[1] * y[1]) + (y[2] * y[2]
                                                       + y[3] * y[3])
                    for perm in bfly:
                        s = s + s.at[perm].get(mode="promise_in_bounds")
                        q = q + q.at[perm].get(mode="promise_in_bounds")
                    for k in range(4):
                        ob[i, pl.ds(k * 16, 16)] = y[k]
                    lj = lanes == j
                    pmv = jnp.where(lj, s, pmv)
                    pvr = jnp.where(lj, q, pvr)
                pmv = pmv * (1.0 / HIDDEN)
                pvr = pvr * (1.0 / HIDDEN) - pmv * pmv
                pin = _rsqrt16(pvr + EPS)
                for j in range(16):
                    i = i0 + j
                    sj = jnp.full((16,), j, jnp.int32)
                    mv = pmv.at[sj].get(mode="promise_in_bounds")
                    inv = pin.at[sj].get(mode="promise_in_bounds")
                    for k in range(4):
                        e = ob[i, pl.ds(k * 16, 16)]
                        ob[i, pl.ds(k * 16, 16)] = ((e - mv) * inv * g[k]
                                                    + b[k])
                return 0

            lax.fori_loop(0, CHUNK // 16, grp_body, 0)

        start_gather(0, 0)

        def outer(co, _):
            for bstat in range(2):
                c = co * 2 + bstat
                nb = 1 - bstat

                @pl.when(c >= 1)
                def _():
                    wait_scatter(nb)

                @pl.when(c + 1 < nchunk)
                def _():
                    start_gather(c + 1, nb)

                wait_gather(bstat)
                compute_chunk(bstat)
                start_scatter(c, bstat)
            return 0

        lax.fori_loop(0, nchunk // 2, outer, 0)
        wait_scatter(1)

    out, _ = sc_kernel(tok2, type_ids.reshape(flat).astype(jnp.int32),
                       token_table, type_table, pos_table, ln_gamma, ln_beta)
    return out


def kernel(token_ids, type_ids, token_table, type_table, pos_table, ln_gamma,
           ln_beta):
    batch, seq = token_ids.shape
    return _run(token_ids, type_ids, token_table, type_table, pos_table,
                ln_gamma, ln_beta, batch=batch, seq=seq)
